# Initial kernel scaffold; baseline (speedup 1.0000x reference)
import jax
import jax.numpy as jnp
from jax.experimental import pallas as pl


def kernel(x, adj, W1, b1, W2, b2):
    raise NotImplementedError("write your pallas kernel here")


# speedup vs baseline: 1.4916x; 1.4916x over previous
import functools

import jax
import jax.numpy as jnp
from jax.experimental import pallas as pl
from jax.experimental.pallas import tpu as pltpu

_N = 10000
_D = 128
_EPS = 0.0


def _body(x_ref, adj_ref, w1_ref, b1_ref, w2_ref, b2_ref, out_ref, *, bm):
    i = pl.program_id(0)
    a = adj_ref[...].astype(jnp.bfloat16)
    xb = x_ref[...].astype(jnp.bfloat16)
    u = jnp.dot(a, xb, preferred_element_type=jnp.float32)

    xr = x_ref[pl.ds(i * bm, bm), :]
    col = jax.lax.broadcasted_iota(jnp.int32, (bm, _D), 1)
    v = jnp.where(col == 0, 0.0, u + (1.0 + _EPS) * xr)
    vn = jnp.maximum(jnp.sqrt(jnp.sum(v * v, axis=1, keepdims=True)), 1e-7)
    e = jnp.exp(vn)
    em = 1.0 / e
    cosh = 0.5 * (e + em)
    sinh_over = 0.5 * (e - em) / vn
    o = jnp.where(col == 0, cosh, sinh_over * v)

    h1 = jnp.maximum(
        jnp.dot(o, w1_ref[...], preferred_element_type=jnp.float32)
        + b1_ref[...], 0.0)
    out_ref[...] = (
        jnp.dot(h1, w2_ref[...], preferred_element_type=jnp.float32)
        + b2_ref[...])


@jax.jit
def kernel(x, adj, W1, b1, W2, b2):
    bm = 400
    grid = (_N // bm,)
    return pl.pallas_call(
        functools.partial(_body, bm=bm),
        grid=grid,
        in_specs=[
            pl.BlockSpec((_N, _D), lambda i: (0, 0)),
            pl.BlockSpec((bm, _N), lambda i: (i, 0)),
            pl.BlockSpec((_D, _D), lambda i: (0, 0)),
            pl.BlockSpec((1, _D), lambda i: (0, 0)),
            pl.BlockSpec((_D, _D), lambda i: (0, 0)),
            pl.BlockSpec((1, _D), lambda i: (0, 0)),
        ],
        out_specs=pl.BlockSpec((bm, _D), lambda i: (i, 0)),
        out_shape=jax.ShapeDtypeStruct((_N, _D), jnp.float32),
        compiler_params=pltpu.CompilerParams(
            dimension_semantics=("arbitrary",)),
    )(x, adj, W1, b1.reshape(1, _D), W2, b2.reshape(1, _D))


# speedup vs baseline: 1.5012x; 1.0064x over previous
import functools

import jax
import jax.numpy as jnp
from jax.experimental import pallas as pl
from jax.experimental.pallas import tpu as pltpu

_N = 10000
_D = 128
_EPS = 0.0


def _body(x_ref, adj_ref, w1_ref, b1_ref, w2_ref, b2_ref, out_ref, *, bm):
    i = pl.program_id(0)
    u = jnp.dot(adj_ref[...], x_ref[...],
                preferred_element_type=jnp.float32,
                precision=jax.lax.Precision.DEFAULT)

    xr = x_ref[pl.ds(i * bm, bm), :]
    col = jax.lax.broadcasted_iota(jnp.int32, (bm, _D), 1)
    v = jnp.where(col == 0, 0.0, u + (1.0 + _EPS) * xr)
    vn = jnp.maximum(jnp.sqrt(jnp.sum(v * v, axis=1, keepdims=True)), 1e-7)
    e = jnp.exp(vn)
    em = 1.0 / e
    cosh = 0.5 * (e + em)
    sinh_over = 0.5 * (e - em) / vn
    o = jnp.where(col == 0, cosh, sinh_over * v)

    h1 = jnp.maximum(
        jnp.dot(o, w1_ref[...], preferred_element_type=jnp.float32)
        + b1_ref[...], 0.0)
    out_ref[...] = (
        jnp.dot(h1, w2_ref[...], preferred_element_type=jnp.float32)
        + b2_ref[...])


@jax.jit
def kernel(x, adj, W1, b1, W2, b2):
    bm = 400
    grid = (_N // bm,)
    return pl.pallas_call(
        functools.partial(_body, bm=bm),
        grid=grid,
        in_specs=[
            pl.BlockSpec((_N, _D), lambda i: (0, 0)),
            pl.BlockSpec((bm, _N), lambda i: (i, 0)),
            pl.BlockSpec((_D, _D), lambda i: (0, 0)),
            pl.BlockSpec((1, _D), lambda i: (0, 0)),
            pl.BlockSpec((_D, _D), lambda i: (0, 0)),
            pl.BlockSpec((1, _D), lambda i: (0, 0)),
        ],
        out_specs=pl.BlockSpec((bm, _D), lambda i: (i, 0)),
        out_shape=jax.ShapeDtypeStruct((_N, _D), jnp.float32),
        compiler_params=pltpu.CompilerParams(
            dimension_semantics=("arbitrary",)),
    )(x, adj, W1, b1.reshape(1, _D), W2, b2.reshape(1, _D))
